# baseline (speedup 1.0000x reference)
import jax
import jax.numpy as jnp
from jax.experimental import pallas as pl
from jax.experimental.pallas import tpu as pltpu

N, E, H, L = 1024, 4096, 64, 2
_CB = 128
_BR = 128
_NBLK = E // _BR

_f32 = jnp.float32


def _combine(s):
    mag = jnp.exp(s[0:1])
    odd = s[1:2] - 2.0 * jnp.floor(s[1:2] * 0.5)
    sign = 1.0 - 2.0 * odd
    return jnp.where(s[2:3] > 0.5, 0.0, mag * sign)


def _fwd(e2t, nb_hbm, src_rr, tar_rr, cave_rr, seed_c, w, ng,
         Wt, bt, Wm1, bm1, Wm2, bm2, Wa1, ba1, Wa2, ba2, Wc, bc,
         W_ih, W_hh, b_ih, b_hh, Ws1, bs1, Ws2, bs2,
         marg_ref, delta_ref, scr, sem, seg_n, seg_e, gat_e):
    relu = jax.nn.relu
    sig = jax.nn.sigmoid
    tar = tar_rr[...]
    cave = cave_rr[...]
    seeds_c = seed_c[...]
    w_r = w[...]
    ng_r = ng[...]

    def dot(a, b):
        return jnp.dot(a, b, preferred_element_type=_f32,
                       precision=jax.lax.Precision.HIGHEST)

    def dot_nt(a, b):
        return jax.lax.dot_general(a, b, (((1,), (1,)), ((), ())),
                                   preferred_element_type=_f32,
                                   precision=jax.lax.Precision.HIGHEST)

    def seg_to(ch, idx_row, out_ref, m_out):
        def body(b, _):
            off = b * _CB
            m = (jax.lax.broadcasted_iota(jnp.int32, (_CB, E), 0) + off
                 == idx_row).astype(_f32)
            out_ref[:, pl.ds(off, _CB)] = dot_nt(ch, m)
            return 0
        jax.lax.fori_loop(0, m_out // _CB, body, 0)
        return out_ref[:, :m_out]

    def segprods(v):
        ch = jnp.concatenate([
            jnp.log(jnp.maximum(jnp.abs(v), 1e-30)),
            (v < 0.0).astype(_f32),
            (v == 0.0).astype(_f32)], axis=0)
        pn = _combine(seg_to(ch, tar, seg_n, N))
        pc = _combine(seg_to(ch, cave, seg_e, E))
        return pn, pc

    def gather_n(rows, k):
        def body(b, _):
            off = b * _CB
            g = (jax.lax.broadcasted_iota(jnp.int32, (N, _CB), 0)
                 == src_rr[:, pl.ds(off, _CB)]).astype(_f32)
            gat_e[:k, pl.ds(off, _CB)] = dot(rows, g)
            return 0
        jax.lax.fori_loop(0, E // _CB, body, 0)
        return gat_e[:k, :]

    def stream_nt(hcat, hbm, rows):
        nblk = rows // _BR
        def start(i):
            cp = pltpu.make_async_copy(
                hbm.at[pl.ds(i * _BR, _BR)], scr.at[i % 2], sem.at[i % 2])
            cp.start()
            return cp
        cps = [start(0), None]
        outs = []
        for i in range(nblk):
            if i + 1 < nblk:
                cps[(i + 1) % 2] = start(i + 1)
            cps[i % 2].wait()
            outs.append(dot_nt(hcat, scr[i % 2]))
        return jnp.concatenate(outs, axis=1)

    seeds = jnp.max(
        (jax.lax.broadcasted_iota(jnp.int32, (32, N), 1) == seeds_c).astype(_f32),
        axis=0, keepdims=True)
    Ps0 = 1.0 - seeds
    Pi0 = seeds
    gath = gather_n(jnp.concatenate([ng_r, Ps0], axis=0), 2)
    gamma = gath[0:1]
    Ps_i0 = gath[1:2]
    Phi0 = 1.0 - Ps_i0
    Theta = 1.0 - w_r * Phi0 + 1e-20

    P_tar, Tc = segprods(Theta)
    mm = gather_n(P_tar, 1) / Tc
    Ps_ij = Ps_i0 * mm
    Phi = (1.0 - w_r) * (1.0 - gamma) * Phi0 - (Ps_ij - Ps_i0)
    Ps_t = Ps0 * P_tar
    Pr_t = ng_r * Pi0
    Pi_t = 1.0 - Ps_t - Pr_t
    m_prev = jnp.concatenate([Ps_t, Pi_t, Pr_t], axis=0)

    def write_marg(t, m3):
        c = jnp.where(m3 <= 0.0, 1e-20, m3)
        c = jnp.where(c > 1.0, 1.0, c)
        marg_ref[t, :, :] = jnp.log(c)

    write_marg(0, m_prev)
    msg = jnp.concatenate([Theta, Phi, Ps_ij], axis=0)
    hid = relu(dot(Wt[...], msg) + bt[...])

    deltas = []
    for t in range(L):
        Theta = Theta - w_r * Phi
        node_msg, Tc = segprods(Theta)
        edge_msg = gather_n(node_msg, 1) / Tc
        msg = jnp.concatenate([Theta, Phi, Ps_ij], axis=0)
        theta_emb = relu(dot(Wt[...], msg) + bt[...])
        nm = relu(dot(Wm1[...], node_msg) + bm1[...])
        em = relu(dot(Wm2[...], edge_msg) + bm2[...])
        hcat = relu(dot(Wc[...], jnp.concatenate([hid, theta_emb], axis=0))
                    + bc[...])
        node_agg = relu(dot(Wa1[...], stream_nt(hcat, e2t, N)) + ba1[...])
        node_res = sig(dot(Ws1[...], jnp.concatenate([node_agg, nm], axis=0))
                       + bs1[...])
        node_scale = node_res[0:1]
        node_delta = node_res[1:2]
        hid_agg = relu(dot(Wa2[...], stream_nt(hcat, nb_hbm, E)) + ba2[...])
        gi = dot(W_ih[...], hid_agg) + b_ih[...]
        gh = dot(W_hh[...], hid) + b_hh[...]
        r = sig(gi[0:H] + gh[0:H])
        z = sig(gi[H:2 * H] + gh[H:2 * H])
        n = jnp.tanh(gi[2 * H:] + r * gh[2 * H:])
        hid = (1.0 - z) * n + z * hid
        edge_res = sig(dot(Ws2[...], jnp.concatenate([hid_agg, em], axis=0))
                       + bs2[...])
        edge_msg = edge_msg * edge_res[0:1] + edge_res[1:2]
        node_msg = jnp.minimum(node_msg * node_scale + node_delta, 1.0)
        Ps_new = Ps0 * node_msg
        Ps_t = jnp.where(Ps_new > Ps_t, Ps_t, Ps_new)
        Pr_new = Pr_t + ng_r * Pi_t
        Pr_t = jnp.where(Pr_new < Pr_t, Pr_t, Pr_new)
        Pi_t = 1.0 - Ps_t - Pr_t
        m_cur = jnp.concatenate([Ps_t, Pi_t, Pr_t], axis=0)
        write_marg(t + 1, m_cur)
        edge_msg = jnp.minimum(edge_msg, 1.0)
        new_Ps_ij = Ps_i0 * edge_msg
        Phi = (1.0 - w_r) * (1.0 - gamma) * Phi - (new_Ps_ij - Ps_ij)
        Ps_ij = new_Ps_ij
        deltas.append(jnp.max(jnp.abs(m_cur - m_prev)).reshape(1, 1))
        m_prev = m_cur

    delta_ref[...] = jnp.concatenate(deltas, axis=1)


def kernel(edge2tnode, nb_matrix, adj_index, cave_index, weights, nodes_gamma,
           seed_list, Wt, bt, Wm1, bm1, Wm2, bm2, Wa1, ba1, Wa2, ba2, Wc, bc,
           W_ih, W_hh, b_ih, b_hh, Ws1, bs1, Ws2, bs2):
    adj = adj_index.astype(jnp.int32)
    src_r = adj[0].reshape(1, E)
    tar_r = adj[1].reshape(1, E)
    cave_r = cave_index.astype(jnp.int32).reshape(1, E)
    seed_c = seed_list.astype(jnp.int32).reshape(32, 1)
    w = weights.astype(_f32).reshape(1, E)
    ng = nodes_gamma.astype(_f32).reshape(1, N)

    vmem = pl.BlockSpec(memory_space=pltpu.VMEM)
    hbm = pl.BlockSpec(memory_space=pl.ANY)
    specs = [hbm, hbm] + [vmem] * 26
    col = lambda b: b.reshape(-1, 1)
    marg, deltas = pl.pallas_call(
        _fwd,
        in_specs=specs,
        out_shape=(jax.ShapeDtypeStruct((L + 1, 3, N), _f32),
                   jax.ShapeDtypeStruct((1, L), _f32)),
        scratch_shapes=[pltpu.VMEM((2, _BR, E), _f32),
                        pltpu.SemaphoreType.DMA((2,)),
                        pltpu.VMEM((3, N), _f32),
                        pltpu.VMEM((3, E), _f32),
                        pltpu.VMEM((2, E), _f32)],
    )(edge2tnode, nb_matrix, src_r, tar_r, cave_r, seed_c, w, ng,
      Wt, col(bt), Wm1, col(bm1), Wm2, col(bm2), Wa1, col(ba1), Wa2, col(ba2),
      Wc, col(bc), W_ih, W_hh, col(b_ih), col(b_hh), Ws1, col(bs1), Ws2, col(bs2))
    return marg.transpose(0, 2, 1), deltas.reshape(L)
